# baseline (speedup 1.0000x reference)
import functools

import jax
import jax.numpy as jnp
from jax import lax
from jax.experimental import pallas as pl
from jax.experimental.pallas import tpu as pltpu
from jax.experimental.pallas import tpu_sc as plsc

N = 50000
E = 800000
R = 8
H = 64
O = 32

NC = 2
NT = 16
HALF = N // NC
PC = 25088
TROWS = PC // NT
NP = NC * PC
DUMP = 25040
ECH = 128
E_PAD = 802816


def _compose1_body(wc_ref, b_ref, o_ref):
    r = pl.program_id(1)
    acc = wc_ref[r, 0] * b_ref[0]
    for b in range(1, R):
        acc = acc + wc_ref[r, b] * b_ref[b]
    o_ref[...] = acc


def _compose1(w_comp1, bases1):
    BN = 1000
    nb = N // BN
    return pl.pallas_call(
        _compose1_body,
        grid=(nb, R),
        in_specs=[
            pl.BlockSpec(memory_space=pltpu.SMEM),
            pl.BlockSpec((R, BN, H), lambda i, r: (0, i, 0)),
        ],
        out_specs=pl.BlockSpec((BN, H), lambda i, r: (r * nb + i, 0)),
        out_shape=jax.ShapeDtypeStruct((R * N, H), jnp.float32),
    )(w_comp1, bases1)


def _dense2_body(h_ref, wc_ref, b2_ref, lw2_ref, hw_ref, l2_ref):
    h = h_ref[...]
    for r in range(R):
        w2r = wc_ref[r, 0] * b2_ref[0]
        for b in range(1, R):
            w2r = w2r + wc_ref[r, b] * b2_ref[b]
        hw_ref[:, r * O:(r + 1) * O] = jnp.dot(
            h, w2r, preferred_element_type=jnp.float32)
    l2_ref[...] = jnp.dot(h, lw2_ref[...], preferred_element_type=jnp.float32)


def _dense2(h_pad, w_comp2, bases2, loop_w2):
    BN = 512
    grid = (NP // BN,)
    return pl.pallas_call(
        _dense2_body,
        grid=grid,
        in_specs=[
            pl.BlockSpec((BN, H), lambda i: (i, 0)),
            pl.BlockSpec(memory_space=pltpu.SMEM),
            pl.BlockSpec((R, H, O), lambda i: (0, 0, 0)),
            pl.BlockSpec((H, O), lambda i: (0, 0)),
        ],
        out_specs=[
            pl.BlockSpec((BN, R * O), lambda i: (i, 0)),
            pl.BlockSpec((BN, O), lambda i: (i, 0)),
        ],
        out_shape=[
            jax.ShapeDtypeStruct((NP, R * O), jnp.float32),
            jax.ShapeDtypeStruct((NP, O), jnp.float32),
        ],
    )(h_pad, w_comp2, bases2, loop_w2)


def _make_sc_scatter(d, loop_is_gather):
    nv = d // 16
    NSLOT = 2 if loop_is_gather else 4
    GE = ECH * NSLOT
    GPT = E_PAD // GE // NT
    WB = 16 if loop_is_gather else 56
    WBN = TROWS // WB
    mesh = plsc.VectorSubcoreMesh(core_axis_name="c", subcore_axis_name="s")

    scratch = {
        "acc": pltpu.VMEM_SHARED((PC, d), jnp.float32),
        "meta_m": [pltpu.VMEM((3, GE), jnp.int32) for _ in range(2)],
        "idx_v": [pltpu.VMEM((ECH,), jnp.int32) for _ in range(NSLOT)],
        "sidx_v": [pltpu.VMEM((ECH,), jnp.int32) for _ in range(NSLOT)],
        "rows_v": [pltpu.VMEM((ECH, d), jnp.float32) for _ in range(NSLOT)],
        "wb_v": pltpu.VMEM((WB, d), jnp.float32),
        "lp_v": pltpu.VMEM((WB, d), jnp.float32),
        "lidx_v": pltpu.VMEM((WB,), jnp.int32),
        "bias_v": pltpu.VMEM((d,), jnp.float32),
        "rsem": [pltpu.SemaphoreType.DMA for _ in range(NSLOT)],
        "ssem": [pltpu.SemaphoreType.DMA for _ in range(NSLOT)],
        "msem": [pltpu.SemaphoreType.DMA for _ in range(2)],
    }
    if loop_is_gather:
        scratch["f_v"] = [[pltpu.VMEM((ECH,), jnp.int32)
                           for _ in range(NSLOT)] for _ in range(2)]
        scratch["fsem"] = [[pltpu.SemaphoreType.DMA
                            for _ in range(NSLOT)] for _ in range(2)]

    @functools.partial(
        pl.kernel,
        mesh=mesh,
        compiler_params=pltpu.CompilerParams(use_tc_tiling_on_sc=False),
        out_type=jax.ShapeDtypeStruct((NP, d), jnp.float32),
        scratch_types=scratch,
    )
    def sc_kernel(table, metap, fraw, lidx, looptab, bias, out, *,
                  acc, meta_m, idx_v, sidx_v, rows_v,
                  wb_v, lp_v, lidx_v, bias_v, rsem, ssem, msem, **fk):
        c = lax.axis_index("c")
        s = lax.axis_index("s")
        lo = c * HALF

        def zrow(i, _):
            for v in range(nv):
                wb_v[i, pl.ds(v * 16, 16)] = jnp.zeros((16,), jnp.float32)
            return 0
        lax.fori_loop(0, WB, zrow, 0)
        for z in range(WBN):
            pltpu.sync_copy(wb_v, acc.at[pl.ds(s * TROWS + z * WB, WB)])
        pltpu.sync_copy(bias, bias_v)
        plsc.subcore_barrier()

        def meta_load(j, p, sync=False):
            base = (s + NT * j) * GE
            src = metap.at[:, pl.ds(base, GE)]
            if sync:
                pltpu.sync_copy(src, meta_m[p])
            else:
                pltpu.async_copy(src, meta_m[p], msem[p])

        def meta_wait(j, p):
            base = (s + NT * j) * GE
            pltpu.make_async_copy(
                metap.at[:, pl.ds(base, GE)], meta_m[p], msem[p]).wait()

        def fgather_start(p):
            for slot in range(NSLOT):
                pltpu.async_copy(
                    fraw.at[meta_m[p].at[0, pl.ds(slot * ECH, ECH)]],
                    fk["f_v"][p][slot], fk["fsem"][p][slot])

        def compute_and_gather(p, slot, j_):
            @pl.when(j_ > 0)
            def _():
                pltpu.make_async_copy(
                    rows_v[slot], acc.at[sidx_v[slot]], ssem[slot]).wait()
            if loop_is_gather:
                pltpu.make_async_copy(
                    fraw.at[meta_m[p].at[0, pl.ds(slot * ECH, ECH)]],
                    fk["f_v"][p][slot], fk["fsem"][p][slot]).wait()
            for v in range(ECH // 16):
                o = slot * ECH + v * 16
                e16 = meta_m[p][1, pl.ds(o, 16)]
                d16 = meta_m[p][2, pl.ds(o, 16)]
                if loop_is_gather:
                    f16 = fk["f_v"][p][slot][pl.ds(v * 16, 16)]
                    idx16 = e16 * N + f16
                else:
                    s16 = meta_m[p][0, pl.ds(o, 16)]
                    rp = jnp.where(s16 >= HALF, s16 + (PC - HALF), s16)
                    idx16 = rp * R + e16
                idx_v[slot][pl.ds(v * 16, 16)] = idx16
                inr = (d16 >= lo) & (d16 < lo + HALF)
                sidx_v[slot][pl.ds(v * 16, 16)] = jnp.where(
                    inr, d16 - lo, DUMP)
            pltpu.async_copy(table.at[idx_v[slot]], rows_v[slot], rsem[slot])

        meta_load(0, 0, sync=True)
        if loop_is_gather:
            fgather_start(0)

        def phased(p_static, j_):
            @pl.when(j_ < GPT - 1)
            def _():
                meta_load(j_ + 1, 1 - p_static)
            for slot in range(NSLOT):
                compute_and_gather(p_static, slot, j_)

            @pl.when(j_ < GPT - 1)
            def _():
                meta_wait(j_ + 1, 1 - p_static)
                if loop_is_gather:
                    fgather_start(1 - p_static)
            for slot in range(NSLOT):
                pltpu.make_async_copy(
                    table.at[idx_v[slot]], rows_v[slot], rsem[slot]).wait()
                pltpu.async_copy(rows_v[slot], acc.at[sidx_v[slot]],
                                 ssem[slot], add=True)

        def egroup(j, _):
            p = lax.rem(j, 2)

            @pl.when(p == 0)
            def _():
                phased(0, j)

            @pl.when(p == 1)
            def _():
                phased(1, j)
            return 0

        lax.fori_loop(0, GPT, egroup, 0)
        for slot in range(NSLOT):
            pltpu.make_async_copy(
                rows_v[slot], acc.at[sidx_v[slot]], ssem[slot]).wait()
        plsc.subcore_barrier()

        def wchunk(z, _):
            r0 = s * TROWS + z * WB
            g0 = c * PC + r0
            pltpu.sync_copy(acc.at[pl.ds(r0, WB)], wb_v)
            if loop_is_gather:
                pltpu.sync_copy(lidx.at[pl.ds(g0, WB)], lidx_v)
                pltpu.async_copy(looptab.at[lidx_v], lp_v, rsem[0]).wait()
            else:
                pltpu.sync_copy(looptab.at[pl.ds(g0, WB)], lp_v)

            def frow(i, _):
                for v in range(nv):
                    x = (wb_v[i, pl.ds(v * 16, 16)]
                         + lp_v[i, pl.ds(v * 16, 16)]
                         + bias_v[pl.ds(v * 16, 16)])
                    wb_v[i, pl.ds(v * 16, 16)] = jnp.maximum(x, 0.0)
                return 0
            lax.fori_loop(0, WB, frow, 0)
            pltpu.sync_copy(wb_v, out.at[pl.ds(g0, WB)])
            return 0
        lax.fori_loop(0, WBN, wchunk, 0)

    return sc_kernel


@functools.lru_cache(maxsize=None)
def _sc_scatter(d, loop_is_gather):
    return _make_sc_scatter(d, loop_is_gather)


def kernel(feat, edge_index, etypes, bases1, w_comp1, loop_w1, bias1,
           bases2, w_comp2, loop_w2, bias2):
    src = edge_index[0]
    dst = edge_index[1]
    zpad = jnp.zeros((PC - HALF,), jnp.int32)
    epad = jnp.zeros((E_PAD - E,), jnp.int32)

    metap = jnp.stack([
        jnp.concatenate([src, epad]),
        jnp.concatenate([etypes, epad]),
        jnp.concatenate([dst, epad - 1]),
    ])
    feat_pad = jnp.concatenate([feat[:HALF], zpad, feat[HALF:], zpad])

    w1_flat = _compose1(w_comp1, bases1)

    h_pad = _sc_scatter(H, True)(w1_flat, metap, feat, feat_pad,
                                 loop_w1, bias1)

    hw_pad, loop2_pad = _dense2(h_pad, w_comp2, bases2, loop_w2)
    hw_flat = hw_pad.reshape(NP * R, O)

    dummy_lidx = jnp.zeros((NP,), jnp.int32)
    out_pad = _sc_scatter(O, False)(hw_flat, metap, feat,
                                    dummy_lidx, loop2_pad, bias2)

    return jnp.concatenate([out_pad[:HALF], out_pad[PC:PC + HALF]], axis=0)


# speedup vs baseline: 1.0360x; 1.0360x over previous
import functools

import jax
import jax.numpy as jnp
from jax import lax
from jax.experimental import pallas as pl
from jax.experimental.pallas import tpu as pltpu
from jax.experimental.pallas import tpu_sc as plsc

N = 50000
E = 800000
R = 8
H = 64
O = 32

NC = 2
NT = 16
HALF = N // NC
PC = 25088
TROWS = PC // NT
NP = NC * PC
DUMP = 25040
ECH = 128
E_PAD = 802816


def _compose1_body(wc_ref, b_ref, o_ref):
    r = pl.program_id(1)
    acc = wc_ref[r, 0] * b_ref[0]
    for b in range(1, R):
        acc = acc + wc_ref[r, b] * b_ref[b]
    o_ref[...] = acc


def _compose1(w_comp1, bases1):
    BN = 1000
    nb = N // BN
    return pl.pallas_call(
        _compose1_body,
        grid=(nb, R),
        in_specs=[
            pl.BlockSpec(memory_space=pltpu.SMEM),
            pl.BlockSpec((R, BN, H), lambda i, r: (0, i, 0)),
        ],
        out_specs=pl.BlockSpec((BN, H), lambda i, r: (r * nb + i, 0)),
        out_shape=jax.ShapeDtypeStruct((R * N, H), jnp.float32),
    )(w_comp1, bases1)


def _dense2_body(a_ref, g_ref, b1_ref, wc_ref, b2_ref, lw2_ref,
                 hw_ref, l2_ref):
    h = jnp.maximum(a_ref[...] + g_ref[...] + b1_ref[...], 0.0)
    for r in range(R):
        w2r = wc_ref[r, 0] * b2_ref[0]
        for b in range(1, R):
            w2r = w2r + wc_ref[r, b] * b2_ref[b]
        hw_ref[:, r * O:(r + 1) * O] = jnp.dot(
            h, w2r, preferred_element_type=jnp.float32)
    l2_ref[...] = jnp.dot(h, lw2_ref[...], preferred_element_type=jnp.float32)


def _dense2(acc1_pad, lpg_pad, bias1, w_comp2, bases2, loop_w2):
    BN = 512
    grid = (NP // BN,)
    return pl.pallas_call(
        _dense2_body,
        grid=grid,
        in_specs=[
            pl.BlockSpec((BN, H), lambda i: (i, 0)),
            pl.BlockSpec((BN, H), lambda i: (i, 0)),
            pl.BlockSpec((1, H), lambda i: (0, 0)),
            pl.BlockSpec(memory_space=pltpu.SMEM),
            pl.BlockSpec((R, H, O), lambda i: (0, 0, 0)),
            pl.BlockSpec((H, O), lambda i: (0, 0)),
        ],
        out_specs=[
            pl.BlockSpec((BN, R * O), lambda i: (i, 0)),
            pl.BlockSpec((BN, O), lambda i: (i, 0)),
        ],
        out_shape=[
            jax.ShapeDtypeStruct((NP, R * O), jnp.float32),
            jax.ShapeDtypeStruct((NP, O), jnp.float32),
        ],
    )(acc1_pad, lpg_pad, bias1.reshape(1, H), w_comp2, bases2, loop_w2)


def _final_body(a_ref, l_ref, b_ref, o_ref):
    o_ref[...] = jnp.maximum(a_ref[...] + l_ref[...] + b_ref[...], 0.0)


def _finalize2(acc2_pad, loop2_pad, bias2):
    BN = 512
    grid = (NP // BN,)
    return pl.pallas_call(
        _final_body,
        grid=grid,
        in_specs=[
            pl.BlockSpec((BN, O), lambda i: (i, 0)),
            pl.BlockSpec((BN, O), lambda i: (i, 0)),
            pl.BlockSpec((1, O), lambda i: (0, 0)),
        ],
        out_specs=pl.BlockSpec((BN, O), lambda i: (i, 0)),
        out_shape=jax.ShapeDtypeStruct((NP, O), jnp.float32),
    )(acc2_pad, loop2_pad, bias2.reshape(1, O))


def _make_sc_scatter(d, loop_is_gather):
    nv = d // 16
    NSLOT = 2 if loop_is_gather else 4
    GE = ECH * NSLOT
    GPT = E_PAD // GE // NT
    LW = 112
    LGN = TROWS // LW
    mesh = plsc.VectorSubcoreMesh(core_axis_name="c", subcore_axis_name="s")

    scratch = {
        "acc": pltpu.VMEM_SHARED((PC, d), jnp.float32),
        "meta_m": [pltpu.VMEM((3, GE), jnp.int32) for _ in range(2)],
        "idx_v": [pltpu.VMEM((ECH,), jnp.int32) for _ in range(NSLOT)],
        "sidx_v": [pltpu.VMEM((ECH,), jnp.int32) for _ in range(NSLOT)],
        "rows_v": [pltpu.VMEM((ECH, d), jnp.float32) for _ in range(NSLOT)],
        "rsem": [pltpu.SemaphoreType.DMA for _ in range(NSLOT)],
        "ssem": [pltpu.SemaphoreType.DMA for _ in range(NSLOT)],
        "msem": [pltpu.SemaphoreType.DMA for _ in range(2)],
    }
    if loop_is_gather:
        scratch["f_v"] = [[pltpu.VMEM((ECH,), jnp.int32)
                           for _ in range(NSLOT)] for _ in range(2)]
        scratch["fsem"] = [[pltpu.SemaphoreType.DMA
                            for _ in range(NSLOT)] for _ in range(2)]
        scratch["lgidx"] = pltpu.VMEM((TROWS,), jnp.int32)
        scratch["lgsem"] = [pltpu.SemaphoreType.DMA for _ in range(2)]
        out_type = [jax.ShapeDtypeStruct((NP, d), jnp.float32),
                    jax.ShapeDtypeStruct((NP, d), jnp.float32)]
    else:
        out_type = jax.ShapeDtypeStruct((NP, d), jnp.float32)

    @functools.partial(
        pl.kernel,
        mesh=mesh,
        compiler_params=pltpu.CompilerParams(use_tc_tiling_on_sc=False),
        out_type=out_type,
        scratch_types=scratch,
    )
    def sc_kernel(*refs, acc, meta_m, idx_v, sidx_v, rows_v,
                  rsem, ssem, msem, **fk):
        if loop_is_gather:
            table, metap, fraw, lidx, looptab, out, lgout = refs
        else:
            table, metap, out = refs
        c = lax.axis_index("c")
        s = lax.axis_index("s")
        lo = c * HALF
        tile0 = s * TROWS
        gl0 = c * PC + tile0

        def zrow(i, _):
            for v in range(nv):
                rows_v[0][i, pl.ds(v * 16, 16)] = jnp.zeros((16,),
                                                            jnp.float32)
            return 0
        lax.fori_loop(0, ECH, zrow, 0)
        for z in range(TROWS // ECH):
            pltpu.sync_copy(rows_v[0], acc.at[pl.ds(tile0 + z * ECH, ECH)])
        rem = TROWS % ECH
        if rem:
            pltpu.sync_copy(rows_v[0].at[pl.ds(0, rem)],
                            acc.at[pl.ds(tile0 + TROWS - rem, rem)])

        if loop_is_gather:
            pltpu.sync_copy(lidx.at[pl.ds(gl0, TROWS)], fk["lgidx"])
            for z in range(LGN):
                q = z % 2
                if z >= 2:
                    pltpu.make_async_copy(
                        rows_v[q].at[pl.ds(0, LW)],
                        lgout.at[pl.ds(gl0 + (z - 2) * LW, LW)],
                        fk["lgsem"][q]).wait()
                pltpu.sync_copy(
                    looptab.at[fk["lgidx"].at[pl.ds(z * LW, LW)]],
                    rows_v[q].at[pl.ds(0, LW)])
                pltpu.async_copy(rows_v[q].at[pl.ds(0, LW)],
                                 lgout.at[pl.ds(gl0 + z * LW, LW)],
                                 fk["lgsem"][q])
            for z in range(LGN - 2, LGN):
                q = z % 2
                pltpu.make_async_copy(
                    rows_v[q].at[pl.ds(0, LW)],
                    lgout.at[pl.ds(gl0 + z * LW, LW)],
                    fk["lgsem"][q]).wait()
        plsc.subcore_barrier()

        def meta_load(j, p, sync=False):
            base = (s + NT * j) * GE
            src = metap.at[:, pl.ds(base, GE)]
            if sync:
                pltpu.sync_copy(src, meta_m[p])
            else:
                pltpu.async_copy(src, meta_m[p], msem[p])

        def meta_wait(j, p):
            base = (s + NT * j) * GE
            pltpu.make_async_copy(
                metap.at[:, pl.ds(base, GE)], meta_m[p], msem[p]).wait()

        def fgather_start(p):
            for slot in range(NSLOT):
                pltpu.async_copy(
                    fraw.at[meta_m[p].at[0, pl.ds(slot * ECH, ECH)]],
                    fk["f_v"][p][slot], fk["fsem"][p][slot])

        def compute_and_gather(p, slot, j_):
            @pl.when(j_ > 0)
            def _():
                pltpu.make_async_copy(
                    rows_v[slot], acc.at[sidx_v[slot]], ssem[slot]).wait()
            if loop_is_gather:
                pltpu.make_async_copy(
                    fraw.at[meta_m[p].at[0, pl.ds(slot * ECH, ECH)]],
                    fk["f_v"][p][slot], fk["fsem"][p][slot]).wait()
            for v in range(ECH // 16):
                o = slot * ECH + v * 16
                e16 = meta_m[p][1, pl.ds(o, 16)]
                d16 = meta_m[p][2, pl.ds(o, 16)]
                if loop_is_gather:
                    f16 = fk["f_v"][p][slot][pl.ds(v * 16, 16)]
                    idx16 = e16 * N + f16
                else:
                    s16 = meta_m[p][0, pl.ds(o, 16)]
                    rp = jnp.where(s16 >= HALF, s16 + (PC - HALF), s16)
                    idx16 = rp * R + e16
                idx_v[slot][pl.ds(v * 16, 16)] = idx16
                inr = (d16 >= lo) & (d16 < lo + HALF)
                sidx_v[slot][pl.ds(v * 16, 16)] = jnp.where(
                    inr, d16 - lo, DUMP)
            pltpu.async_copy(table.at[idx_v[slot]], rows_v[slot], rsem[slot])

        meta_load(0, 0, sync=True)
        if loop_is_gather:
            fgather_start(0)

        def phased(p_static, j_):
            @pl.when(j_ < GPT - 1)
            def _():
                meta_load(j_ + 1, 1 - p_static)
            for slot in range(NSLOT):
                compute_and_gather(p_static, slot, j_)

            @pl.when(j_ < GPT - 1)
            def _():
                meta_wait(j_ + 1, 1 - p_static)
                if loop_is_gather:
                    fgather_start(1 - p_static)
            for slot in range(NSLOT):
                pltpu.make_async_copy(
                    table.at[idx_v[slot]], rows_v[slot], rsem[slot]).wait()
                pltpu.async_copy(rows_v[slot], acc.at[sidx_v[slot]],
                                 ssem[slot], add=True)

        def egroup(j, _):
            p = lax.rem(j, 2)

            @pl.when(p == 0)
            def _():
                phased(0, j)

            @pl.when(p == 1)
            def _():
                phased(1, j)
            return 0

        lax.fori_loop(0, GPT, egroup, 0)
        for slot in range(NSLOT):
            pltpu.make_async_copy(
                rows_v[slot], acc.at[sidx_v[slot]], ssem[slot]).wait()
        plsc.subcore_barrier()

        pltpu.sync_copy(acc.at[pl.ds(tile0, TROWS)],
                        out.at[pl.ds(gl0, TROWS)])

    return sc_kernel


@functools.lru_cache(maxsize=None)
def _sc_scatter(d, loop_is_gather):
    return _make_sc_scatter(d, loop_is_gather)


def kernel(feat, edge_index, etypes, bases1, w_comp1, loop_w1, bias1,
           bases2, w_comp2, loop_w2, bias2):
    src = edge_index[0]
    dst = edge_index[1]
    zpad = jnp.zeros((PC - HALF,), jnp.int32)
    epad = jnp.zeros((E_PAD - E,), jnp.int32)

    metap = jnp.stack([
        jnp.concatenate([src, epad]),
        jnp.concatenate([etypes, epad]),
        jnp.concatenate([dst, epad - 1]),
    ])
    feat_pad = jnp.concatenate([feat[:HALF], zpad, feat[HALF:], zpad])

    w1_flat = _compose1(w_comp1, bases1)

    acc1_pad, lpg_pad = _sc_scatter(H, True)(w1_flat, metap, feat, feat_pad,
                                             loop_w1)

    hw_pad, loop2_pad = _dense2(acc1_pad, lpg_pad, bias1,
                                w_comp2, bases2, loop_w2)
    hw_flat = hw_pad.reshape(NP * R, O)

    acc2_pad = _sc_scatter(O, False)(hw_flat, metap)
    out_pad = _finalize2(acc2_pad, loop2_pad, bias2)

    return jnp.concatenate([out_pad[:HALF], out_pad[PC:PC + HALF]], axis=0)


# speedup vs baseline: 1.2257x; 1.1831x over previous
import functools

import jax
import jax.numpy as jnp
from jax import lax
from jax.experimental import pallas as pl
from jax.experimental.pallas import tpu as pltpu
from jax.experimental.pallas import tpu_sc as plsc

N = 50000
E = 800000
R = 8
H = 64
O = 32

NC = 2
NT = 16
HALF = N // NC
PC = 25088
TROWS = PC // NT
NP = NC * PC
DUMP = 25040
ECH = 128
E_PAD = 802816


def _compose1_body(wc_ref, b_ref, o_ref):
    for r in range(R):
        acc = wc_ref[r, 0] * b_ref[0]
        for b in range(1, R):
            acc = acc + wc_ref[r, b] * b_ref[b]
        o_ref[r] = acc


def _compose1(w_comp1, bases1):
    bases1_r = bases1.reshape(R, N * H // 128, 128)
    BN = 1000
    grid = (bases1_r.shape[1] // BN,)
    w1 = pl.pallas_call(
        _compose1_body,
        grid=grid,
        in_specs=[
            pl.BlockSpec(memory_space=pltpu.SMEM),
            pl.BlockSpec((R, BN, 128), lambda i: (0, i, 0)),
        ],
        out_specs=pl.BlockSpec((R, BN, 128), lambda i: (0, i, 0)),
        out_shape=jax.ShapeDtypeStruct(bases1_r.shape, jnp.float32),
    )(w_comp1, bases1_r)
    return w1.reshape(R * N, H)


def _dense2_body(a_ref, g_ref, b1_ref, wc_ref, b2_ref, lw2_ref,
                 hw_ref, l2_ref):
    h = jnp.maximum(a_ref[...] + g_ref[...] + b1_ref[...], 0.0)
    for r in range(R):
        w2r = wc_ref[r, 0] * b2_ref[0]
        for b in range(1, R):
            w2r = w2r + wc_ref[r, b] * b2_ref[b]
        hw_ref[:, r * O:(r + 1) * O] = jnp.dot(
            h, w2r, preferred_element_type=jnp.float32)
    l2_ref[...] = jnp.dot(h, lw2_ref[...], preferred_element_type=jnp.float32)


def _dense2(acc1_pad, lpg_pad, bias1, w_comp2, bases2, loop_w2):
    BN = 512
    grid = (NP // BN,)
    return pl.pallas_call(
        _dense2_body,
        grid=grid,
        in_specs=[
            pl.BlockSpec((BN, H), lambda i: (i, 0)),
            pl.BlockSpec((BN, H), lambda i: (i, 0)),
            pl.BlockSpec((1, H), lambda i: (0, 0)),
            pl.BlockSpec(memory_space=pltpu.SMEM),
            pl.BlockSpec((R, H, O), lambda i: (0, 0, 0)),
            pl.BlockSpec((H, O), lambda i: (0, 0)),
        ],
        out_specs=[
            pl.BlockSpec((BN, R * O), lambda i: (i, 0)),
            pl.BlockSpec((BN, O), lambda i: (i, 0)),
        ],
        out_shape=[
            jax.ShapeDtypeStruct((NP, R * O), jnp.float32),
            jax.ShapeDtypeStruct((NP, O), jnp.float32),
        ],
    )(acc1_pad, lpg_pad, bias1.reshape(1, H), w_comp2, bases2, loop_w2)


def _final_body(a_ref, l_ref, b_ref, o_ref):
    o_ref[...] = jnp.maximum(a_ref[...] + l_ref[...] + b_ref[...], 0.0)


def _finalize2(acc2_pad, loop2_pad, bias2):
    BN = 512
    grid = (NP // BN,)
    return pl.pallas_call(
        _final_body,
        grid=grid,
        in_specs=[
            pl.BlockSpec((BN, O), lambda i: (i, 0)),
            pl.BlockSpec((BN, O), lambda i: (i, 0)),
            pl.BlockSpec((1, O), lambda i: (0, 0)),
        ],
        out_specs=pl.BlockSpec((BN, O), lambda i: (i, 0)),
        out_shape=jax.ShapeDtypeStruct((NP, O), jnp.float32),
    )(acc2_pad, loop2_pad, bias2.reshape(1, O))


def _make_sc_scatter(d, loop_is_gather):
    nv = d // 16
    NSLOT = 2 if loop_is_gather else 4
    GE = ECH * NSLOT
    GPT = E_PAD // GE // NT
    LW = 112
    LGN = TROWS // LW
    mesh = plsc.VectorSubcoreMesh(core_axis_name="c", subcore_axis_name="s")

    scratch = {
        "acc": pltpu.VMEM_SHARED((PC, d), jnp.float32),
        "meta_m": [[pltpu.VMEM((GE,), jnp.int32) for _ in range(3)]
                   for _ in range(2)],
        "idx_v": [pltpu.VMEM((ECH,), jnp.int32) for _ in range(NSLOT)],
        "sidx_v": [pltpu.VMEM((ECH,), jnp.int32) for _ in range(NSLOT)],
        "rows_v": [pltpu.VMEM((ECH, d), jnp.float32) for _ in range(NSLOT)],
        "rsem": [pltpu.SemaphoreType.DMA for _ in range(NSLOT)],
        "ssem": [pltpu.SemaphoreType.DMA for _ in range(NSLOT)],
        "msem": [pltpu.SemaphoreType.DMA for _ in range(2)],
    }
    if loop_is_gather:
        scratch["f_v"] = [[pltpu.VMEM((ECH,), jnp.int32)
                           for _ in range(NSLOT)] for _ in range(2)]
        scratch["fsem"] = [[pltpu.SemaphoreType.DMA
                            for _ in range(NSLOT)] for _ in range(2)]
        scratch["lgidx"] = pltpu.VMEM((TROWS,), jnp.int32)
        scratch["lgsem"] = [pltpu.SemaphoreType.DMA for _ in range(2)]
        out_type = [jax.ShapeDtypeStruct((NP, d), jnp.float32),
                    jax.ShapeDtypeStruct((NP, d), jnp.float32)]
    else:
        out_type = jax.ShapeDtypeStruct((NP, d), jnp.float32)

    @functools.partial(
        pl.kernel,
        mesh=mesh,
        compiler_params=pltpu.CompilerParams(use_tc_tiling_on_sc=False),
        out_type=out_type,
        scratch_types=scratch,
    )
    def sc_kernel(*refs, acc, meta_m, idx_v, sidx_v, rows_v,
                  rsem, ssem, msem, **fk):
        if loop_is_gather:
            srcp, etp, dstp, table, fraw, lidx, looptab, out, lgout = refs
        else:
            srcp, etp, dstp, table, out = refs
        metas = (srcp, etp, dstp)
        c = lax.axis_index("c")
        s = lax.axis_index("s")
        lo = c * HALF
        tile0 = s * TROWS
        gl0 = c * PC + tile0

        def zrow(i, _):
            for v in range(nv):
                rows_v[0][i, pl.ds(v * 16, 16)] = jnp.zeros((16,),
                                                            jnp.float32)
            return 0
        lax.fori_loop(0, ECH, zrow, 0)
        for z in range(TROWS // ECH):
            pltpu.sync_copy(rows_v[0], acc.at[pl.ds(tile0 + z * ECH, ECH)])
        rem = TROWS % ECH
        if rem:
            pltpu.sync_copy(rows_v[0].at[pl.ds(0, rem)],
                            acc.at[pl.ds(tile0 + TROWS - rem, rem)])

        if loop_is_gather:
            pltpu.sync_copy(lidx.at[pl.ds(gl0, TROWS)], fk["lgidx"])
            for z in range(LGN):
                q = z % 2
                if z >= 2:
                    pltpu.make_async_copy(
                        rows_v[q].at[pl.ds(0, LW)],
                        lgout.at[pl.ds(gl0 + (z - 2) * LW, LW)],
                        fk["lgsem"][q]).wait()
                pltpu.sync_copy(
                    looptab.at[fk["lgidx"].at[pl.ds(z * LW, LW)]],
                    rows_v[q].at[pl.ds(0, LW)])
                pltpu.async_copy(rows_v[q].at[pl.ds(0, LW)],
                                 lgout.at[pl.ds(gl0 + z * LW, LW)],
                                 fk["lgsem"][q])
            for z in range(LGN - 2, LGN):
                q = z % 2
                pltpu.make_async_copy(
                    rows_v[q].at[pl.ds(0, LW)],
                    lgout.at[pl.ds(gl0 + z * LW, LW)],
                    fk["lgsem"][q]).wait()
        plsc.subcore_barrier()

        def meta_load(j, p, sync=False):
            base = (s + NT * j) * GE
            for hbm, buf in zip(metas, meta_m[p]):
                if sync:
                    pltpu.sync_copy(hbm.at[pl.ds(base, GE)], buf)
                else:
                    pltpu.async_copy(hbm.at[pl.ds(base, GE)], buf, msem[p])

        def meta_wait(j, p):
            base = (s + NT * j) * GE
            for hbm, buf in zip(metas, meta_m[p]):
                pltpu.make_async_copy(
                    hbm.at[pl.ds(base, GE)], buf, msem[p]).wait()

        def fgather_start(p):
            for slot in range(NSLOT):
                pltpu.async_copy(
                    fraw.at[meta_m[p][0].at[pl.ds(slot * ECH, ECH)]],
                    fk["f_v"][p][slot], fk["fsem"][p][slot])

        def compute_and_gather(p, slot, j_):
            @pl.when(j_ > 0)
            def _():
                pltpu.make_async_copy(
                    rows_v[slot], acc.at[sidx_v[slot]], ssem[slot]).wait()
            if loop_is_gather:
                pltpu.make_async_copy(
                    fraw.at[meta_m[p][0].at[pl.ds(slot * ECH, ECH)]],
                    fk["f_v"][p][slot], fk["fsem"][p][slot]).wait()
            for v in range(ECH // 16):
                o = slot * ECH + v * 16
                e16 = meta_m[p][1][pl.ds(o, 16)]
                d16 = meta_m[p][2][pl.ds(o, 16)]
                if loop_is_gather:
                    f16 = fk["f_v"][p][slot][pl.ds(v * 16, 16)]
                    idx16 = e16 * N + f16
                else:
                    s16 = meta_m[p][0][pl.ds(o, 16)]
                    rp = jnp.where(s16 >= HALF, s16 + (PC - HALF), s16)
                    idx16 = rp * R + e16
                idx_v[slot][pl.ds(v * 16, 16)] = idx16
                inr = (d16 >= lo) & (d16 < lo + HALF)
                sidx_v[slot][pl.ds(v * 16, 16)] = jnp.where(
                    inr, d16 - lo, DUMP)
            pltpu.async_copy(table.at[idx_v[slot]], rows_v[slot], rsem[slot])

        meta_load(0, 0, sync=True)
        if loop_is_gather:
            fgather_start(0)

        def phased(p_static, j_):
            @pl.when(j_ < GPT - 1)
            def _():
                meta_load(j_ + 1, 1 - p_static)
            for slot in range(NSLOT):
                compute_and_gather(p_static, slot, j_)

            @pl.when(j_ < GPT - 1)
            def _():
                meta_wait(j_ + 1, 1 - p_static)
                if loop_is_gather:
                    fgather_start(1 - p_static)
            for slot in range(NSLOT):
                pltpu.make_async_copy(
                    table.at[idx_v[slot]], rows_v[slot], rsem[slot]).wait()
                pltpu.async_copy(rows_v[slot], acc.at[sidx_v[slot]],
                                 ssem[slot], add=True)

        def egroup(j, _):
            p = lax.rem(j, 2)

            @pl.when(p == 0)
            def _():
                phased(0, j)

            @pl.when(p == 1)
            def _():
                phased(1, j)
            return 0

        lax.fori_loop(0, GPT, egroup, 0)
        for slot in range(NSLOT):
            pltpu.make_async_copy(
                rows_v[slot], acc.at[sidx_v[slot]], ssem[slot]).wait()
        plsc.subcore_barrier()

        pltpu.sync_copy(acc.at[pl.ds(tile0, TROWS)],
                        out.at[pl.ds(gl0, TROWS)])

    return sc_kernel


@functools.lru_cache(maxsize=None)
def _sc_scatter(d, loop_is_gather):
    return _make_sc_scatter(d, loop_is_gather)


def kernel(feat, edge_index, etypes, bases1, w_comp1, loop_w1, bias1,
           bases2, w_comp2, loop_w2, bias2):
    src = edge_index[0]
    dst = edge_index[1]
    zpad = jnp.zeros((PC - HALF,), jnp.int32)
    epad = jnp.zeros((E_PAD - E,), jnp.int32)

    srcp = jnp.concatenate([src, epad])
    etp = jnp.concatenate([etypes, epad])
    dstp = jnp.concatenate([dst, epad - 1])
    feat_pad = jnp.concatenate([feat[:HALF], zpad, feat[HALF:], zpad])

    w1_flat = _compose1(w_comp1, bases1)

    acc1_pad, lpg_pad = _sc_scatter(H, True)(srcp, etp, dstp, w1_flat,
                                             feat, feat_pad, loop_w1)

    hw_pad, loop2_pad = _dense2(acc1_pad, lpg_pad, bias1,
                                w_comp2, bases2, loop_w2)
    hw_flat = hw_pad.reshape(NP * R, O)

    acc2_pad = _sc_scatter(O, False)(srcp, etp, dstp, hw_flat)
    out_pad = _finalize2(acc2_pad, loop2_pad, bias2)

    return jnp.concatenate([out_pad[:HALF], out_pad[PC:PC + HALF]], axis=0)
